# Initial kernel scaffold; baseline (speedup 1.0000x reference)
import jax
import jax.numpy as jnp
from jax.experimental import pallas as pl


def kernel(x, Wg, bg, W1, b1, W2, b2, Ws1, bs1, Ws2, bs2, bias):
    raise NotImplementedError("write your pallas kernel here")


# speedup vs baseline: 1.3561x; 1.3561x over previous
import functools

import jax
import jax.numpy as jnp
from jax.experimental import pallas as pl
from jax.experimental.pallas import tpu as pltpu

E = 23
K = 3
D = 768
H = 384
EP = 128
BT = 128
NT = 72
NEG = -1e30


def _gelu(v):
    return 0.5 * v * (1.0 + jax.lax.erf(v * 0.7071067811865476))


def _gate_kernel(x_ref, wg_ref, bg_ref, bias_ref,
                 topw_ref, pos_ref, counts_ref, aux_ref):
    T = x_ref.shape[0]
    x = x_ref[...]
    logits = jax.lax.dot_general(x, wg_ref[...], (((1,), (0,)), ((), ())),
                                 preferred_element_type=jnp.float32)
    gw = jax.nn.sigmoid(logits + bg_ref[...])
    lane = jax.lax.broadcasted_iota(jnp.int32, (T, EP), 1)
    valid = lane < E
    gwm = jnp.where(valid, gw, 0.0)
    sel = jnp.where(valid, gw + bias_ref[...], NEG)

    masks, vals = [], []
    for _ in range(K):
        m = jnp.max(sel, axis=1, keepdims=True)
        ismax = sel == m
        idx = jnp.min(jnp.where(ismax, lane, EP), axis=1, keepdims=True)
        onek = lane == idx
        masks.append(onek)
        vals.append(jnp.sum(jnp.where(onek, gwm, 0.0), axis=1, keepdims=True))
        sel = jnp.where(onek, NEG, sel)

    wsum = vals[0] + vals[1] + vals[2]
    o3 = (masks[0].astype(jnp.float32) + masks[1].astype(jnp.float32)
          + masks[2].astype(jnp.float32))
    counts_f = jnp.sum(o3, axis=0, keepdims=True)

    CB = 256
    tri = (jax.lax.broadcasted_iota(jnp.int32, (CB, CB), 0)
           > jax.lax.broadcasted_iota(jnp.int32, (CB, CB), 1)).astype(jnp.float32)
    carry = jnp.zeros((1, EP), jnp.float32)
    rows = []
    for b in range(T // CB):
        ob = o3[b * CB:(b + 1) * CB]
        rows.append(jax.lax.dot_general(tri, ob, (((1,), (0,)), ((), ())),
                                        preferred_element_type=jnp.float32) + carry)
        carry = carry + jnp.sum(ob, axis=0, keepdims=True)
    cnt_before = jnp.concatenate(rows, axis=0)

    ntiles = jnp.floor((counts_f + (BT - 1)) / BT)
    triu = (jax.lax.broadcasted_iota(jnp.int32, (EP, EP), 0)
            < jax.lax.broadcasted_iota(jnp.int32, (EP, EP), 1)).astype(jnp.float32)
    poffset = BT * jax.lax.dot_general(ntiles, triu, (((1,), (0,)), ((), ())),
                                       preferred_element_type=jnp.float32)

    slot_f = cnt_before + poffset
    topw = jnp.zeros((T, EP), jnp.float32)
    pos = jnp.zeros((T, EP), jnp.float32)
    for k in range(K):
        pos_k = jnp.sum(jnp.where(masks[k], slot_f, 0.0), axis=1, keepdims=True)
        topw = topw + jnp.where(lane == k, vals[k] / wsum, 0.0)
        pos = pos + jnp.where(lane == k, pos_k, 0.0)

    topw_ref[...] = topw
    pos_ref[...] = pos.astype(jnp.int32)
    counts_ref[...] = counts_f.astype(jnp.int32)

    gwn = gwm / jnp.sum(gwm, axis=1, keepdims=True)
    Pv = jnp.sum(gwn, axis=0, keepdims=True) / T
    Fv = E * counts_f / (K * T)
    aux_ref[...] = jnp.sum(Pv * Fv, keepdims=True)


def _ffn_kernel(te_ref, xs_ref, w1_ref, b1_ref, w2_ref, b2_ref, ys_ref):
    x = xs_ref[...]
    h = jax.lax.dot_general(x, w1_ref[...][0], (((1,), (1,)), ((), ())),
                            preferred_element_type=jnp.float32) + b1_ref[...][0]
    h = _gelu(h)
    y = jax.lax.dot_general(h, w2_ref[...][0], (((1,), (1,)), ((), ())),
                            preferred_element_type=jnp.float32) + b2_ref[...][0]
    ys_ref[...] = y


def _combine_kernel(x_ref, yg_ref, tw_ref, ws1_ref, bs1_ref, ws2_ref, bs2_ref,
                    o_ref):
    x = x_ref[...]
    h = jax.lax.dot_general(x, ws1_ref[...], (((1,), (1,)), ((), ())),
                            preferred_element_type=jnp.float32) + bs1_ref[...]
    h = _gelu(h)
    acc = jax.lax.dot_general(h, ws2_ref[...], (((1,), (1,)), ((), ())),
                              preferred_element_type=jnp.float32) + bs2_ref[...]
    tw = tw_ref[...]
    yg = yg_ref[...]
    for k in range(K):
        acc = acc + tw[:, k:k + 1] * yg[:, k * D:(k + 1) * D]
    o_ref[...] = acc


def kernel(x, Wg, bg, W1, b1, W2, b2, Ws1, bs1, Ws2, bs2, bias):
    o_shape = x.shape
    x2 = x.reshape(-1, D)
    T = x2.shape[0]
    PADN = NT * BT

    wg_p = jnp.zeros((D, EP), Wg.dtype).at[:, :E].set(Wg.T)
    bg_p = jnp.zeros((1, EP), bg.dtype).at[0, :E].set(bg)
    bias_p = jnp.zeros((1, EP), bias.dtype).at[0, :E].set(bias)

    topw, pos, counts, aux = pl.pallas_call(
        _gate_kernel,
        out_shape=[
            jax.ShapeDtypeStruct((T, EP), jnp.float32),
            jax.ShapeDtypeStruct((T, EP), jnp.int32),
            jax.ShapeDtypeStruct((1, EP), jnp.int32),
            jax.ShapeDtypeStruct((1, 1), jnp.float32),
        ],
    )(x2, wg_p, bg_p, bias_p)

    cnt = counts[0, :E]
    ntiles = (cnt + BT - 1) // BT
    tile_expert = jnp.repeat(jnp.arange(E, dtype=jnp.int32), ntiles,
                             total_repeat_length=NT)

    posf = pos[:, :K].reshape(-1)
    inv = jnp.zeros((PADN,), jnp.int32).at[posf].set(
        jnp.arange(T * K, dtype=jnp.int32) // K)
    xs = x2[inv]

    grid_spec = pltpu.PrefetchScalarGridSpec(
        num_scalar_prefetch=1,
        grid=(NT,),
        in_specs=[
            pl.BlockSpec((BT, D), lambda j, te: (j, 0)),
            pl.BlockSpec((1, H, D), lambda j, te: (te[j], 0, 0)),
            pl.BlockSpec((1, 1, H), lambda j, te: (te[j], 0, 0)),
            pl.BlockSpec((1, D, H), lambda j, te: (te[j], 0, 0)),
            pl.BlockSpec((1, 1, D), lambda j, te: (te[j], 0, 0)),
        ],
        out_specs=pl.BlockSpec((BT, D), lambda j, te: (j, 0)),
    )
    ys = pl.pallas_call(
        _ffn_kernel,
        grid_spec=grid_spec,
        out_shape=jax.ShapeDtypeStruct((PADN, D), jnp.float32),
    )(tile_expert, xs, W1, b1.reshape(E, 1, H), W2, b2.reshape(E, 1, D))

    yg = ys[posf].reshape(T, K * D)

    BTC = 256
    out = pl.pallas_call(
        _combine_kernel,
        grid=(T // BTC,),
        in_specs=[
            pl.BlockSpec((BTC, D), lambda i: (i, 0)),
            pl.BlockSpec((BTC, K * D), lambda i: (i, 0)),
            pl.BlockSpec((BTC, EP), lambda i: (i, 0)),
            pl.BlockSpec((H, D), lambda i: (0, 0)),
            pl.BlockSpec((1, H), lambda i: (0, 0)),
            pl.BlockSpec((D, H), lambda i: (0, 0)),
            pl.BlockSpec((1, D), lambda i: (0, 0)),
        ],
        out_specs=pl.BlockSpec((BTC, D), lambda i: (i, 0)),
        out_shape=jax.ShapeDtypeStruct((T, D), jnp.float32),
    )(x2, yg, topw, Ws1, bs1.reshape(1, H), Ws2, bs2.reshape(1, D))

    return out.reshape(o_shape), aux[0, 0]
